# baseline (speedup 1.0000x reference)
import functools

import jax
import jax.numpy as jnp
from jax import lax
from jax.experimental import pallas as pl
from jax.experimental.pallas import tpu as pltpu
from jax.experimental.pallas import tpu_sc as plsc

_EMBED_DIM = 32
_BATCH = 16384
_LANES = 16
_PASS = 256
_NSEM = 8


def _gather_body(b_per_w,
                 user_hbm, item_hbm, uidx_hbm, iidx_hbm,
                 uout_hbm, iout_hbm,
                 uidx_v, iidx_v, urows_v, irows_v, *sems):
    sem_u = sems[:_NSEM]
    sem_i = sems[_NSEM:]
    sid = lax.axis_index("s")
    wid = sid * 2 + lax.axis_index("c")
    base = wid * b_per_w
    pltpu.sync_copy(uidx_hbm.at[pl.ds(base, b_per_w)], uidx_v)
    pltpu.sync_copy(iidx_hbm.at[pl.ds(base, b_per_w)], iidx_v)

    ngroups = _PASS // _LANES

    def enqueue(p, g):
        src_off = p * _PASS + g * _LANES
        dst_off = g * _LANES
        uvec = uidx_v[pl.ds(src_off, _LANES)]
        ivec = iidx_v[pl.ds(src_off, _LANES)]
        for j in range(_LANES):
            pltpu.async_copy(user_hbm.at[pl.ds(uvec[j], 1)],
                             urows_v.at[pl.ds(dst_off + j, 1)],
                             sem_u[j % _NSEM])
            pltpu.async_copy(item_hbm.at[pl.ds(ivec[j], 1)],
                             irows_v.at[pl.ds(dst_off + j, 1)],
                             sem_i[j % _NSEM])

    def drain(g):
        for j in range(_LANES):
            pltpu.make_async_copy(
                user_hbm.at[pl.ds(0, 1)],
                urows_v.at[pl.ds(g * _LANES + j, 1)],
                sem_u[j % _NSEM]).wait()
            pltpu.make_async_copy(
                item_hbm.at[pl.ds(0, 1)],
                irows_v.at[pl.ds(g * _LANES + j, 1)],
                sem_i[j % _NSEM]).wait()

    for p in range(b_per_w // _PASS):
        enqueue(p, 0)
        enqueue(p, 1)

        def step(g, _):
            enqueue(p, g)
            drain(g - 2)
            return 0

        lax.fori_loop(2, ngroups, step, 0)
        drain(ngroups - 2)
        drain(ngroups - 1)
        pltpu.sync_copy(urows_v, uout_hbm.at[pl.ds(base + p * _PASS, _PASS)])
        pltpu.sync_copy(irows_v, iout_hbm.at[pl.ds(base + p * _PASS, _PASS)])


def kernel(user_table, item_table, user_idx, item_idx):
    info = plsc.get_sparse_core_info()
    nw = info.num_cores * info.num_subcores
    b_per_w = _BATCH // nw

    mesh = plsc.VectorSubcoreMesh(core_axis_name="c", subcore_axis_name="s")
    out_type = (
        jax.ShapeDtypeStruct((_BATCH, _EMBED_DIM), jnp.float32),
        jax.ShapeDtypeStruct((_BATCH, _EMBED_DIM), jnp.float32),
    )
    scratch = [
        pltpu.VMEM((b_per_w,), jnp.int32),
        pltpu.VMEM((b_per_w,), jnp.int32),
        pltpu.VMEM((_PASS, _EMBED_DIM), jnp.float32),
        pltpu.VMEM((_PASS, _EMBED_DIM), jnp.float32),
    ] + [pltpu.SemaphoreType.DMA] * (2 * _NSEM)
    body = functools.partial(_gather_body, b_per_w)
    run = pl.kernel(body, mesh=mesh, out_type=out_type,
                    scratch_types=scratch)
    return run(user_table, item_table,
               user_idx.astype(jnp.int32), item_idx.astype(jnp.int32))


# speedup vs baseline: 1.0932x; 1.0932x over previous
import functools

import jax
import jax.numpy as jnp
from jax import lax
from jax.experimental import pallas as pl
from jax.experimental.pallas import tpu as pltpu
from jax.experimental.pallas import tpu_sc as plsc

_EMBED_DIM = 32
_BATCH = 16384
_LANES = 16
_PASS = 256


def _gather_body(b_per_w,
                 user_hbm, item_hbm, uidx_hbm, iidx_hbm,
                 uout_hbm, iout_hbm,
                 uidx_v, iidx_v, urows_v, irows_v, sem_u, sem_i):
    sid = lax.axis_index("s")
    wid = sid * 2 + lax.axis_index("c")
    base = wid * b_per_w
    pltpu.sync_copy(uidx_hbm.at[pl.ds(base, b_per_w)], uidx_v)
    pltpu.sync_copy(iidx_hbm.at[pl.ds(base, b_per_w)], iidx_v)

    ngroups = _PASS // _LANES

    def enqueue(p, g):
        src_off = p * _PASS + g * _LANES
        dst_off = g * _LANES
        uvec = uidx_v[pl.ds(src_off, _LANES)]
        ivec = iidx_v[pl.ds(src_off, _LANES)]
        for j in range(_LANES):
            pltpu.async_copy(user_hbm.at[pl.ds(uvec[j], 1)],
                             urows_v.at[pl.ds(dst_off + j, 1)], sem_u)
            pltpu.async_copy(item_hbm.at[pl.ds(ivec[j], 1)],
                             irows_v.at[pl.ds(dst_off + j, 1)], sem_i)

    def drain(g):
        pltpu.make_async_copy(
            user_hbm.at[pl.ds(0, _LANES)],
            urows_v.at[pl.ds(g * _LANES, _LANES)], sem_u).wait()
        pltpu.make_async_copy(
            item_hbm.at[pl.ds(0, _LANES)],
            irows_v.at[pl.ds(g * _LANES, _LANES)], sem_i).wait()

    for p in range(b_per_w // _PASS):
        enqueue(p, 0)
        enqueue(p, 1)

        def step(g, _):
            enqueue(p, g)
            drain(g - 2)
            return 0

        lax.fori_loop(2, ngroups, step, 0)
        drain(ngroups - 2)
        drain(ngroups - 1)
        pltpu.sync_copy(urows_v, uout_hbm.at[pl.ds(base + p * _PASS, _PASS)])
        pltpu.sync_copy(irows_v, iout_hbm.at[pl.ds(base + p * _PASS, _PASS)])


def kernel(user_table, item_table, user_idx, item_idx):
    info = plsc.get_sparse_core_info()
    nw = info.num_cores * info.num_subcores
    b_per_w = _BATCH // nw

    mesh = plsc.VectorSubcoreMesh(core_axis_name="c", subcore_axis_name="s")
    out_type = (
        jax.ShapeDtypeStruct((_BATCH, _EMBED_DIM), jnp.float32),
        jax.ShapeDtypeStruct((_BATCH, _EMBED_DIM), jnp.float32),
    )
    scratch = [
        pltpu.VMEM((b_per_w,), jnp.int32),
        pltpu.VMEM((b_per_w,), jnp.int32),
        pltpu.VMEM((_PASS, _EMBED_DIM), jnp.float32),
        pltpu.VMEM((_PASS, _EMBED_DIM), jnp.float32),
        pltpu.SemaphoreType.DMA,
        pltpu.SemaphoreType.DMA,
    ]
    body = functools.partial(_gather_body, b_per_w)
    run = pl.kernel(body, mesh=mesh, out_type=out_type,
                    scratch_types=scratch)
    return run(user_table, item_table,
               user_idx.astype(jnp.int32), item_idx.astype(jnp.int32))
